# Initial kernel scaffold; baseline (speedup 1.0000x reference)
import jax
import jax.numpy as jnp
from jax.experimental import pallas as pl


def kernel(z, embedding_weight):
    raise NotImplementedError("write your pallas kernel here")


# speedup vs baseline: 1.3280x; 1.3280x over previous
import functools

import jax
import jax.numpy as jnp
from jax import lax
from jax.experimental import pallas as pl
from jax.experimental.pallas import tpu as pltpu
from jax.experimental.pallas import tpu_sc as plsc

_CB = 8192
_D = 64
_BETA = 0.25

_BM = 1024
_BN = 4096
_NE = _CB // _BN


def _argmin_body(z_ref, z2_ref, e_ref, e2_ref, idx_ref, psum_ref,
                 mv, mi, sel):
    j = pl.program_id(1)

    mm = lax.dot_general(z_ref[...], e_ref[...], (((1,), (1,)), ((), ())),
                         preferred_element_type=jnp.float32)
    s = (z2_ref[...] + e2_ref[...]) - 2.0 * mm
    tmin = jnp.min(s, axis=1, keepdims=True)
    ii = lax.broadcasted_iota(jnp.int32, s.shape, 1)
    targ = (jnp.min(jnp.where(s == tmin, ii, _BN), axis=1, keepdims=True)
            + j * _BN)

    @pl.when(j == 0)
    def _():
        sel[...] = tmin
        mi[...] = targ
        mv[...] = tmin.astype(jnp.bfloat16).astype(jnp.float32)

    @pl.when(j > 0)
    def _():
        lt = tmin < mv[...]
        mi[...] = jnp.where(lt, targ, mi[...])
        sel[...] = jnp.where(lt, tmin, sel[...])
        mv[...] = jnp.where(lt, tmin, mv[...]).astype(
            jnp.bfloat16).astype(jnp.float32)

    @pl.when(j == _NE - 1)
    def _():
        idx_ref[...] = mi[...]
        psum_ref[0, 0, 0] = jnp.sum(sel[...])


def _argmin_call(z_flat, z2, e, e2):
    nf = z_flat.shape[0]
    nz = nf // _BM
    return pl.pallas_call(
        _argmin_body,
        grid=(nz, _NE),
        in_specs=[
            pl.BlockSpec((_BM, _D), lambda i, j: (i, 0)),
            pl.BlockSpec((_BM, 1), lambda i, j: (i, 0)),
            pl.BlockSpec((_BN, _D), lambda i, j: (j, 0)),
            pl.BlockSpec((1, _BN), lambda i, j: (0, j)),
        ],
        out_specs=[
            pl.BlockSpec((_BM, 1), lambda i, j: (i, 0)),
            pl.BlockSpec((1, 1, 1), lambda i, j: (i, 0, 0),
                         memory_space=pltpu.SMEM),
        ],
        out_shape=[
            jax.ShapeDtypeStruct((nf, 1), jnp.int32),
            jax.ShapeDtypeStruct((nz, 1, 1), jnp.float32),
        ],
        scratch_shapes=[
            pltpu.VMEM((_BM, 1), jnp.float32),
            pltpu.VMEM((_BM, 1), jnp.int32),
            pltpu.VMEM((_BM, 1), jnp.float32),
        ],
        compiler_params=pltpu.CompilerParams(
            dimension_semantics=("parallel", "arbitrary")),
    )(z_flat, z2, e, e2)


def _gather_call(table_pad, idx_flat):
    nf = idx_flat.shape[0]
    dp = table_pad.shape[1]
    info = plsc.get_sparse_core_info()
    nw = info.num_cores * info.num_subcores
    bpw = nf // nw
    mesh = plsc.VectorSubcoreMesh(core_axis_name="c", subcore_axis_name="s")

    @functools.partial(
        pl.kernel, mesh=mesh,
        out_type=jax.ShapeDtypeStruct((nf, dp), jnp.float32),
        scratch_types=[
            pltpu.VMEM((bpw,), jnp.int32),
            pltpu.VMEM((bpw, dp), jnp.float32),
            pltpu.SemaphoreType.DMA,
        ],
    )
    def k(table_hbm, idx_hbm, out_hbm, idx_v, rows_v, sem):
        wid = lax.axis_index("s") * info.num_cores + lax.axis_index("c")
        base = wid * bpw
        pltpu.sync_copy(idx_hbm.at[pl.ds(base, bpw)], idx_v)
        pltpu.async_copy(table_hbm.at[idx_v], rows_v, sem).wait()
        pltpu.sync_copy(rows_v, out_hbm.at[pl.ds(base, bpw)])

    return k(table_pad, idx_flat)


def kernel(z, embedding_weight):
    z_flat = z.reshape(-1, _D)
    nf = z_flat.shape[0]
    z2 = jnp.sum(z_flat ** 2, axis=1)[:, None]
    e2 = jnp.sum(embedding_weight ** 2, axis=1)[None, :]

    idx2d, psum = _argmin_call(z_flat, z2, embedding_weight, e2)
    idx_flat = idx2d.reshape(nf)

    table_pad = jnp.pad(embedding_weight, ((0, 0), (0, 128 - _D)))
    zq_pad = _gather_call(table_pad, idx_flat)

    loss = (1.0 + _BETA) * jnp.sum(psum) / (nf * _D)
    z_q_st = zq_pad[:, :_D].reshape(z.shape)
    indices = idx_flat.reshape(z.shape[:-1])
    return (z_q_st, loss, indices)
